# Initial kernel scaffold; baseline (speedup 1.0000x reference)
import jax
import jax.numpy as jnp
from jax.experimental import pallas as pl


def kernel(queries, sample_t, rays_d, point_xyz, point_feat, point_rgb, W1, b1, Wa, ba, Wc, bc):
    raise NotImplementedError("write your pallas kernel here")


# speedup vs baseline: 1.3532x; 1.3532x over previous
import functools

import jax
import jax.numpy as jnp
from jax import lax
from jax.experimental import pallas as pl
from jax.experimental.pallas import tpu as pltpu
from jax.experimental.pallas import tpu_sc as plsc

R, S, K, KNN, DF = 1024, 4, 100000, 8, 32
Q = R * S
TP = 2048
KPAD = 100352
NPT = KPAD // TP
BQ = 256
NQB = Q // BQ
QK = Q * KNN
D_TAB = 48
BQ3 = 512
IMAX = 2147483647


def _topk_body(q_ref, pt_ref, idx_ref, d2_ref, rkey_ref, rgidx_ref):
    pt = pl.program_id(1)

    @pl.when(pt == 0)
    def _init():
        rkey_ref[...] = jnp.full((BQ, KNN), IMAX, jnp.int32)
        rgidx_ref[...] = jnp.zeros((BQ, KNN), jnp.int32)

    q = q_ref[...]
    p = pt_ref[...]
    q2 = jnp.sum(q * q, axis=1, keepdims=True)
    p2 = jnp.sum(p * p, axis=0, keepdims=True)
    qp = jnp.dot(q.astype(jnp.bfloat16), p.astype(jnp.bfloat16),
                 preferred_element_type=jnp.float32)
    d2 = (q2 - 2.0 * qp) + p2

    lane = lax.broadcasted_iota(jnp.int32, (1, TP), 1)
    b = lax.bitcast_convert_type(d2, jnp.int32)
    key = b ^ ((b >> 31) & 0x7FFFFFFF)

    kcols, icols = [], []
    for _ in range(KNN):
        m = jnp.min(key, axis=1, keepdims=True)
        pos = jnp.min(jnp.where(key == m, lane, IMAX), axis=1, keepdims=True)
        kcols.append(m)
        icols.append(pos)
        key = jnp.where(lane == pos, IMAX, key)
    t8k = jnp.concatenate(kcols, axis=1)
    t8i = jnp.concatenate(icols, axis=1) + pt * TP

    comb_k = jnp.concatenate([rkey_ref[...], t8k], axis=1)
    comb_g = jnp.concatenate([rgidx_ref[...], t8i], axis=1)
    i16 = lax.broadcasted_iota(jnp.int32, (1, 2 * KNN), 1)
    nk, ng = [], []
    for _ in range(KNN):
        m = jnp.min(comb_k, axis=1, keepdims=True)
        pos = jnp.min(jnp.where(comb_k == m, i16, 99),
                      axis=1, keepdims=True)
        sel = i16 == pos
        nk.append(m)
        ng.append(jnp.min(jnp.where(sel, comb_g, IMAX), axis=1, keepdims=True))
        comb_k = jnp.where(sel, IMAX, comb_k)
    rkey_ref[...] = jnp.concatenate(nk, axis=1)
    rgidx_ref[...] = jnp.concatenate(ng, axis=1)

    @pl.when(pt == NPT - 1)
    def _flush():
        idx_ref[...] = rgidx_ref[...]
        kk = rkey_ref[...]
        d2_ref[...] = lax.bitcast_convert_type(
            kk ^ ((kk >> 31) & 0x7FFFFFFF), jnp.float32)


def _topk(queries, p_t):
    return pl.pallas_call(
        _topk_body,
        grid=(NQB, NPT),
        in_specs=[
            pl.BlockSpec((BQ, 3), lambda qb, pt: (qb, 0)),
            pl.BlockSpec((3, TP), lambda qb, pt: (0, pt)),
        ],
        out_specs=[pl.BlockSpec((BQ, KNN), lambda qb, pt: (qb, 0)),
                   pl.BlockSpec((BQ, KNN), lambda qb, pt: (qb, 0))],
        out_shape=[jax.ShapeDtypeStruct((Q, KNN), jnp.int32),
                   jax.ShapeDtypeStruct((Q, KNN), jnp.float32)],
        scratch_shapes=[pltpu.VMEM((BQ, KNN), jnp.int32),
                        pltpu.VMEM((BQ, KNN), jnp.int32)],
    )(queries, p_t)


def _make_sc_gather():
    info = plsc.get_sparse_core_info()
    nw = info.num_cores * info.num_subcores
    b_per_w = QK // nw
    mesh = plsc.VectorSubcoreMesh(core_axis_name="c", subcore_axis_name="s")

    @functools.partial(
        pl.kernel, mesh=mesh,
        compiler_params=pltpu.CompilerParams(use_tc_tiling_on_sc=False),
        out_type=jax.ShapeDtypeStruct((QK, D_TAB), jnp.float32),
        scratch_types=[
            pltpu.VMEM((b_per_w,), jnp.int32),
            pltpu.VMEM((b_per_w, D_TAB), jnp.float32),
            pltpu.SemaphoreType.DMA,
        ],
    )
    def gather_k(table_hbm, idx_hbm, out_hbm, idx_v, rows_v, sem):
        wid = lax.axis_index("s") * info.num_cores + lax.axis_index("c")
        base = wid * b_per_w
        pltpu.sync_copy(idx_hbm.at[pl.ds(base, b_per_w)], idx_v)
        pltpu.async_copy(table_hbm.at[idx_v], rows_v, sem).wait()
        pltpu.sync_copy(rows_v, out_hbm.at[pl.ds(base, b_per_w)])

    return gather_k


def _sc_gather(table, flat_idx):
    return _make_sc_gather()(table, flat_idx)


def _mlp_body(g_ref, qrep_ref, d2_ref, rays_ref, w1_ref, b1_ref, wa_ref,
              ba_ref, wc_ref, bc_ref, sigma_ref, color_ref):
    g = g_ref[...]
    qrep = qrep_ref[...]
    nb_feat = g[:, :DF]
    nb_xyz = g[:, DF:DF + 3]
    nb_rgb = g[:, DF + 3:DF + 6]
    rel = nb_xyz - qrep
    d2 = d2_ref[...]
    dist = jnp.sqrt(jnp.maximum(d2, 1e-12))
    w = 1.0 / (dist + 1e-8)

    feats = jnp.concatenate([nb_feat, nb_rgb, rel], axis=1)
    h = jnp.dot(feats.astype(jnp.bfloat16), w1_ref[...].astype(jnp.bfloat16),
                preferred_element_type=jnp.float32)
    h = jnp.maximum(h + b1_ref[...], 0.0)

    wh = (w * h).reshape(BQ3, KNN, 64)
    agg = jnp.sum(wh, axis=1)
    wsum = jnp.sum(w.reshape(BQ3, KNN, 1), axis=1)
    agg = agg / wsum

    sp = jnp.dot(agg.astype(jnp.bfloat16), wa_ref[...].astype(jnp.bfloat16),
                 preferred_element_type=jnp.float32)
    sp = sp + ba_ref[...]
    sigma_ref[...] = jnp.maximum(sp, 0.0) + jnp.log(1.0 + jnp.exp(-jnp.abs(sp)))

    rays = rays_ref[...]
    dirs = rays / (jnp.sqrt(jnp.sum(rays * rays, axis=1, keepdims=True)) + 1e-8)
    cin = jnp.concatenate([agg, dirs], axis=1)
    cl = jnp.dot(cin.astype(jnp.bfloat16), wc_ref[...].astype(jnp.bfloat16),
                 preferred_element_type=jnp.float32)
    cl = cl + bc_ref[...]
    color_ref[...] = 1.0 / (1.0 + jnp.exp(-cl))


def _mlp(rows, qrep, d2sel, raysrep, w1, b1, wa, ba, wc, bc):
    full = lambda shape: pl.BlockSpec(shape, lambda i: (0, 0))
    return pl.pallas_call(
        _mlp_body,
        grid=(Q // BQ3,),
        in_specs=[
            pl.BlockSpec((BQ3 * KNN, D_TAB), lambda i: (i, 0)),
            pl.BlockSpec((BQ3 * KNN, 3), lambda i: (i, 0)),
            pl.BlockSpec((BQ3 * KNN, 1), lambda i: (i, 0)),
            pl.BlockSpec((BQ3, 3), lambda i: (i, 0)),
            full((DF + 6, 64)), full((1, 64)), full((64, 1)), full((1, 1)),
            full((64 + 3, 3)), full((1, 3)),
        ],
        out_specs=[pl.BlockSpec((BQ3, 1), lambda i: (i, 0)),
                   pl.BlockSpec((BQ3, 3), lambda i: (i, 0))],
        out_shape=[jax.ShapeDtypeStruct((Q, 1), jnp.float32),
                   jax.ShapeDtypeStruct((Q, 3), jnp.float32)],
    )(rows, qrep, d2sel, raysrep, w1, b1, wa, ba, wc, bc)


def _render_body(t_ref, sig_ref, col_ref, out_ref):
    t = jnp.clip(t_ref[...], 0.0, 1e6)
    c = [t[:, i:i + 1] for i in range(S)]

    def mnmx(a, b):
        return jnp.minimum(a, b), jnp.maximum(a, b)

    c0, c1 = mnmx(c[0], c[1])
    c2, c3 = mnmx(c[2], c[3])
    c0, c2 = mnmx(c0, c2)
    c1, c3 = mnmx(c1, c3)
    c1, c2 = mnmx(c1, c2)

    deltas = [c1 - c0, c2 - c1, c3 - c2, jnp.full((R, 1), 10.0, jnp.float32)]
    sig = sig_ref[...]
    alphas = [1.0 - jnp.exp(-sig[:, i:i + 1] * deltas[i]) for i in range(S)]
    trans = jnp.ones((R, 1), jnp.float32)
    col = col_ref[...]
    rgb = jnp.zeros((R, 3), jnp.float32)
    wsum = jnp.zeros((R, 1), jnp.float32)
    for i in range(S):
        w_i = alphas[i] * trans
        rgb = rgb + w_i * col[:, 3 * i:3 * i + 3]
        wsum = wsum + w_i
        trans = trans * (1.0 - alphas[i] + 1e-10)
    out_ref[...] = rgb + (1.0 - wsum)


def _render(sample_t, sig_rs, col_rs):
    return pl.pallas_call(
        _render_body,
        out_shape=jax.ShapeDtypeStruct((R, 3), jnp.float32),
    )(sample_t, sig_rs, col_rs)


def kernel(queries, sample_t, rays_d, point_xyz, point_feat, point_rgb,
           W1, b1, Wa, ba, Wc, bc):
    pad = jnp.full((KPAD - K, 3), 1e6, jnp.float32)
    p_t = jnp.concatenate([point_xyz, pad], axis=0).T
    idx, d2sel = _topk(queries, p_t)

    table = jnp.concatenate(
        [point_feat, point_xyz, point_rgb,
         jnp.zeros((K, D_TAB - DF - 6), jnp.float32)], axis=1)
    rows = _sc_gather(table, idx.reshape(-1))

    qrep = jnp.repeat(queries, KNN, axis=0)
    raysrep = jnp.repeat(rays_d, S, axis=0)
    sigma, color = _mlp(rows, qrep, d2sel.reshape(QK, 1), raysrep,
                        W1, b1.reshape(1, 64), Wa, ba.reshape(1, 1),
                        Wc, bc.reshape(1, 3))
    return _render(sample_t, sigma.reshape(R, S), color.reshape(R, S * 3))
